# baseline (speedup 1.0000x reference)
import functools

import jax
import jax.numpy as jnp
from jax.experimental import pallas as pl
from jax.experimental.pallas import tpu as pltpu

_NA = 3
_NC = 80
_C = _NC + 5
_AW = (10.0, 16.0, 33.0)
_AH = (13.0, 30.0, 23.0)


def _yolo_body(stride_ref, x_ref, o_ref, *, g, schunk):
    i = pl.program_id(0)
    stride = stride_ref[0, 0]
    v = x_ref[...]
    sh = v.shape
    s = jax.nn.sigmoid(v)
    e = jnp.exp(v)
    cid = jax.lax.broadcasted_iota(jnp.int32, (1, 1, sh[2]), 2)
    rid = jax.lax.rem(cid, _C)
    sid = i * schunk + jax.lax.broadcasted_iota(jnp.int32, (sh[0], 1, 1), 0)
    gx = jax.lax.rem(sid, g).astype(jnp.float32)
    gy = jax.lax.div(sid, g).astype(jnp.float32)
    grid_off = jnp.where(rid == 0, gx, gy)
    aw = jnp.where(cid < _C, _AW[0], jnp.where(cid < 2 * _C, _AW[1], _AW[2]))
    ah = jnp.where(cid < _C, _AH[0], jnp.where(cid < 2 * _C, _AH[1], _AH[2]))
    anch = jnp.where(rid == 2, aw, ah)
    box01 = (s + grid_off) * stride
    box23 = e * anch
    w = jnp.where(rid < 2, box01, jnp.where(rid < 4, box23, s))
    o_ref[:, :, 0:_NA * _C] = w
    o_ref[:, :, _NA * _C:] = jnp.zeros((sh[0], sh[1], 1), jnp.float32)


def kernel(x, img_dim):
    B = x.shape[0]
    g = x.shape[2]
    hw = g * g
    nc = _NA * _C
    stride = (jnp.asarray(img_dim, jnp.float32) / g).reshape(1, 1)
    xt = jnp.transpose(x, (2, 3, 0, 1)).reshape(hw, B, nc)
    schunk = hw // 26
    out = pl.pallas_call(
        functools.partial(_yolo_body, g=g, schunk=schunk),
        grid=(hw // schunk,),
        in_specs=[
            pl.BlockSpec(memory_space=pltpu.SMEM),
            pl.BlockSpec((schunk, B, nc), lambda i: (i, 0, 0)),
        ],
        out_specs=pl.BlockSpec((schunk, B, nc + 1), lambda i: (i, 0, 0)),
        out_shape=jax.ShapeDtypeStruct((hw, B, nc + 1), jnp.float32),
    )(stride, xt)
    o = out[:, :, :nc].reshape(hw, B, _NA, _C)
    return o.transpose(1, 2, 0, 3).reshape(B, _NA * hw, _C)


# speedup vs baseline: 2.7062x; 2.7062x over previous
import functools

import jax
import jax.numpy as jnp
from jax.experimental import pallas as pl
from jax.experimental.pallas import tpu as pltpu

_NA = 3
_NC = 80
_C = _NC + 5
_CB = 5
_AW = (10.0, 16.0, 33.0)
_AH = (13.0, 30.0, 23.0)


def _yolo_body(stride_ref, x0_ref, x1_ref, x2_ref, o_ref, *, g):
    c0 = pl.program_id(0) * _CB
    stride = stride_ref[0, 0]
    hw = g * g
    for a, x_ref in enumerate((x0_ref, x1_ref, x2_ref)):
        v = x_ref[...]
        s = jax.nn.sigmoid(v)
        rid = c0 + jax.lax.broadcasted_iota(jnp.int32, v.shape, 0)
        lane = jax.lax.broadcasted_iota(jnp.int32, v.shape, 2)
        gx = jax.lax.rem(lane, g).astype(jnp.float32)
        gy = jax.lax.div(lane, g).astype(jnp.float32)
        grid_off = jnp.where(rid == 0, gx, gy)
        anch = jnp.where(rid == 2, _AW[a], _AH[a])
        box01 = (s + grid_off) * stride
        box23 = jnp.exp(v) * anch
        w = jnp.where(rid < 2, box01, jnp.where(rid < 4, box23, s))
        o_ref[:, :, a * hw:(a + 1) * hw] = w


def kernel(x, img_dim):
    B = x.shape[0]
    g = x.shape[2]
    hw = g * g
    stride = (jnp.asarray(img_dim, jnp.float32) / g).reshape(1, 1)
    y = x.reshape(B, _NA * _C, hw).transpose(1, 0, 2)
    nblk = _C // _CB
    bblk = B // 2
    out = pl.pallas_call(
        functools.partial(_yolo_body, g=g),
        grid=(nblk, B // bblk),
        in_specs=[
            pl.BlockSpec(memory_space=pltpu.SMEM),
            pl.BlockSpec((_CB, bblk, hw), lambda i, j: (i, j, 0)),
            pl.BlockSpec((_CB, bblk, hw), lambda i, j: (nblk + i, j, 0)),
            pl.BlockSpec((_CB, bblk, hw), lambda i, j: (2 * nblk + i, j, 0)),
        ],
        out_specs=pl.BlockSpec((_CB, bblk, _NA * hw), lambda i, j: (i, j, 0)),
        out_shape=jax.ShapeDtypeStruct((_C, B, _NA * hw), jnp.float32),
    )(stride, y, y, y)
    return jnp.transpose(out, (1, 2, 0))
